# baseline (speedup 1.0000x reference)
import jax
import jax.numpy as jnp
from jax.experimental import pallas as pl

TEMP = 50.0


def _vq_kernel(x0_ref, x1_ref, x2_ref, x3_ref, c_ref, o_ref):
    xrefs = (x0_ref, x1_ref, x2_ref, x3_ref)
    c = c_ref[:]
    cnorm = jnp.sum(c * c, axis=1, keepdims=True)
    caug = jnp.concatenate(
        [c, jnp.ones((c.shape[0], 1), jnp.float32)], axis=1
    )
    lane = jax.lax.broadcasted_iota(jnp.int32, (1, 8192), 1)
    rem = lane & 3
    for r in range(8):
        logits = (-TEMP) * cnorm
        for d in range(4):
            xd = xrefs[d][r : r + 1, :]
            logits = logits + (2.0 * TEMP) * c[:, d : d + 1] * xd
        m = jnp.max(logits, axis=0, keepdims=True)
        e = jnp.exp(logits - m)
        w = jax.lax.dot_general(
            caug,
            e,
            (((0,), (0,)), ((), ())),
            preferred_element_type=jnp.float32,
        )
        inv = 1.0 / w[4:5, :]
        out = jnp.zeros((1, 8192), jnp.float32)
        for d in range(4):
            od = w[d : d + 1, :] * inv
            od_rep = jnp.repeat(od, 4, axis=1)
            out = jnp.where(rem == d, od_rep, out)
        o_ref[r : r + 1, :] = out


def kernel(x, center):
    B, F = x.shape
    n2 = F // 4
    xs = [x[:, d::4] for d in range(4)]
    out = pl.pallas_call(
        _vq_kernel,
        grid=(1,),
        in_specs=[pl.BlockSpec((B, n2), lambda i: (0, 0)) for _ in range(4)]
        + [pl.BlockSpec((512, 4), lambda i: (0, 0))],
        out_specs=pl.BlockSpec((B, F), lambda i: (0, 0)),
        out_shape=jax.ShapeDtypeStruct((B, F), jnp.float32),
    )(*xs, center)
    return out


# speedup vs baseline: 1.9932x; 1.9932x over previous
import jax
import jax.numpy as jnp
from jax.experimental import pallas as pl

TEMP = 50.0
BN = 2048


def _vq_kernel(xt_ref, c_ref, o_ref):
    xt = xt_ref[:]
    c = c_ref[:]
    cnorm = jnp.sum(c * c, axis=1, keepdims=True)
    logits = (-TEMP) * cnorm + (2.0 * TEMP) * c[:, 0:1] * xt[0:1, :]
    for d in range(1, 4):
        logits = logits + (2.0 * TEMP) * c[:, d : d + 1] * xt[d : d + 1, :]
    m = jnp.max(logits, axis=0, keepdims=True)
    e = jnp.exp(logits - m)
    caug = jnp.concatenate(
        [c, jnp.ones((c.shape[0], 1), jnp.float32)], axis=1
    )
    w = jax.lax.dot_general(
        caug,
        e,
        (((0,), (0,)), ((), ())),
        preferred_element_type=jnp.float32,
    )
    o_ref[:] = w[0:4, :] / w[4:5, :]


def kernel(x, center):
    B, F = x.shape
    n = (B * F) // 4
    xt = x.reshape(n, 4).T
    grid = n // BN
    ot = pl.pallas_call(
        _vq_kernel,
        grid=(grid,),
        in_specs=[
            pl.BlockSpec((4, BN), lambda i: (0, i)),
            pl.BlockSpec((512, 4), lambda i: (0, 0)),
        ],
        out_specs=pl.BlockSpec((4, BN), lambda i: (0, i)),
        out_shape=jax.ShapeDtypeStruct((4, n), jnp.float32),
    )(xt, center)
    return ot.T.reshape(B, F)


# speedup vs baseline: 5.4322x; 2.7254x over previous
import jax
import jax.numpy as jnp
from jax.experimental import pallas as pl

TEMP = 50.0


def _vq_kernel(x_ref, c_ref, o_ref):
    c = c_ref[:]
    cnorm = jnp.sum(c * c, axis=1, keepdims=True)
    caug = jnp.concatenate(
        [c, jnp.ones((c.shape[0], 1), jnp.float32)], axis=1
    )
    xt = x_ref[:].reshape(512, 128).T
    outs = []
    for g in range(32):
        logits = (-TEMP) * cnorm + (2.0 * TEMP) * c[:, 0:1] * xt[4 * g : 4 * g + 1, :]
        for d in range(1, 4):
            logits = logits + (2.0 * TEMP) * c[:, d : d + 1] * xt[4 * g + d : 4 * g + d + 1, :]
        m = jnp.max(logits, axis=0, keepdims=True)
        e = jnp.exp(logits - m)
        w = jax.lax.dot_general(
            caug,
            e,
            (((0,), (0,)), ((), ())),
            preferred_element_type=jnp.float32,
        )
        outs.append(w[0:4, :] / w[4:5, :])
    out = jnp.concatenate(outs, axis=0)
    o_ref[:] = out.T.reshape(8, 8192)


def kernel(x, center):
    B, F = x.shape
    out = pl.pallas_call(
        _vq_kernel,
        grid=(1,),
        in_specs=[
            pl.BlockSpec((B, F), lambda i: (0, 0)),
            pl.BlockSpec((512, 4), lambda i: (0, 0)),
        ],
        out_specs=pl.BlockSpec((B, F), lambda i: (0, 0)),
        out_shape=jax.ShapeDtypeStruct((B, F), jnp.float32),
    )(x, center)
    return out


# speedup vs baseline: 6.2465x; 1.1499x over previous
import jax
import jax.numpy as jnp
from jax.experimental import pallas as pl

TEMP = 50.0


def _vq_kernel(x_ref, c_ref, o_ref):
    c = c_ref[:]
    cnorm = jnp.sum(c * c, axis=1, keepdims=True)
    caug = jnp.concatenate(
        [c, jnp.ones((c.shape[0], 1), jnp.float32)], axis=1
    )
    inv_ln2 = 1.4426950408889634
    c2 = (2.0 * TEMP * inv_ln2) * c
    bias2 = (-TEMP * inv_ln2) * cnorm
    ch = c2.astype(jnp.bfloat16).astype(jnp.float32)
    cl = c2 - ch
    bh = bias2.astype(jnp.bfloat16).astype(jnp.float32)
    bl = bias2 - bh
    amat = jnp.concatenate([ch, ch, cl, bh, bl], axis=1)
    ones2 = jnp.ones((2, 512), jnp.float32)
    xt = x_ref[:].reshape(512, 128).T
    outs = []
    for g in range(32):
        x4 = xt[4 * g : 4 * g + 4, :]
        xh = x4.astype(jnp.bfloat16).astype(jnp.float32)
        xl = x4 - xh
        bmat = jnp.concatenate([xh, xl, xh, ones2], axis=0)
        logits = jax.lax.dot_general(
            amat,
            bmat,
            (((1,), (0,)), ((), ())),
            preferred_element_type=jnp.float32,
        )
        m = jnp.max(logits, axis=0, keepdims=True)
        e = jnp.exp2(logits - m)
        w = jax.lax.dot_general(
            caug,
            e,
            (((0,), (0,)), ((), ())),
            preferred_element_type=jnp.float32,
        )
        outs.append(w[0:4, :] / w[4:5, :])
    out = jnp.concatenate(outs, axis=0)
    o_ref[:] = out.T.reshape(8, 8192)


def kernel(x, center):
    B, F = x.shape
    out = pl.pallas_call(
        _vq_kernel,
        grid=(1,),
        in_specs=[
            pl.BlockSpec((B, F), lambda i: (0, 0)),
            pl.BlockSpec((512, 4), lambda i: (0, 0)),
        ],
        out_specs=pl.BlockSpec((B, F), lambda i: (0, 0)),
        out_shape=jax.ShapeDtypeStruct((B, F), jnp.float32),
    )(x, center)
    return out
